# baseline (speedup 1.0000x reference)
import functools

import jax
import jax.numpy as jnp
from jax import lax
from jax.experimental import pallas as pl
from jax.experimental.pallas import tpu as pltpu
from jax.experimental.pallas import tpu_sc as plsc

L1_CAP, L2_CAP, L3_CAP = 2048, 4096, 8192
SDR, CDIM = 2048, 384
N = 2048

_BLK = 1024
_SC_NC = 2
_SC_NS = 16
_NW = _SC_NC * _SC_NS
_PER = 64


def _copy_body(a, o):
    o[...] = a[...]


def _concat_body(split, a, b, o):
    i = pl.program_id(0)

    @pl.when(i < split)
    def _():
        o[...] = a[...]

    @pl.when(i >= split)
    def _():
        o[...] = b[...]


def _tc_copy(a):
    rows = a.shape[0]
    spec = pl.BlockSpec((_BLK, SDR), lambda i: (i, 0))
    return pl.pallas_call(
        _copy_body,
        grid=(rows // _BLK,),
        in_specs=[spec],
        out_specs=spec,
        out_shape=jax.ShapeDtypeStruct((rows, SDR), jnp.float32),
    )(a)


def _tc_concat(a, b, rows, a_rows, b_row0):
    split = a_rows // _BLK
    boff = b_row0 // _BLK
    return pl.pallas_call(
        functools.partial(_concat_body, split),
        grid=(rows // _BLK,),
        in_specs=[
            pl.BlockSpec((_BLK, SDR), lambda i: (jnp.minimum(i, split - 1), 0)),
            pl.BlockSpec((_BLK, SDR),
                         lambda i: (jnp.maximum(i, split) - split + boff, 0)),
        ],
        out_specs=pl.BlockSpec((_BLK, SDR), lambda i: (i, 0)),
        out_shape=jax.ShapeDtypeStruct((rows, SDR), jnp.float32),
    )(a, b)


def _sc_body(contents, l1c, l2c, l3c, o1c, o2c, o3c, b0, b1, sin, sout):
    w = lax.axis_index("s") * _SC_NC + lax.axis_index("c")
    off = w * _PER
    regs = [
        (contents, 0, o1c, 0),
        (l1c, 0, o2c, 0),
        (l2c, N, o2c, N),
        (l2c, 0, o3c, 0),
        (l3c, N, o3c, N),
        (l3c, N + _NW * _PER, o3c, N + _NW * _PER),
        (l3c, N + 2 * _NW * _PER, o3c, N + 2 * _NW * _PER),
    ]
    bufs = (b0, b1)
    n = len(regs)
    cins, couts = [], []
    for i, (src, s0, dst, d0) in enumerate(regs):
        buf = bufs[i % 2]
        cins.append(pltpu.make_async_copy(
            src.at[pl.ds(s0 + off, _PER)], buf, sin.at[i % 2]))
        couts.append(pltpu.make_async_copy(
            buf, dst.at[pl.ds(d0 + off, _PER)], sout.at[i % 2]))
    cins[0].start()
    for i in range(n):
        cins[i].wait()
        couts[i].start()
        if i + 1 < n:
            if i >= 1:
                couts[i - 1].wait()
            cins[i + 1].start()
    couts[n - 2].wait()
    couts[n - 1].wait()


def _sc_content_copy(contents, l1c, l2c, l3c):
    mesh = plsc.VectorSubcoreMesh(
        core_axis_name="c", subcore_axis_name="s",
        num_cores=_SC_NC, num_subcores=_SC_NS)
    f = pl.kernel(
        _sc_body,
        out_type=[
            jax.ShapeDtypeStruct((L1_CAP, CDIM), jnp.float32),
            jax.ShapeDtypeStruct((L2_CAP, CDIM), jnp.float32),
            jax.ShapeDtypeStruct((L3_CAP, CDIM), jnp.float32),
        ],
        mesh=mesh,
        scratch_types=[
            pltpu.VMEM((_PER, CDIM), jnp.float32),
            pltpu.VMEM((_PER, CDIM), jnp.float32),
            pltpu.SemaphoreType.DMA((2,)),
            pltpu.SemaphoreType.DMA((2,)),
        ],
    )
    return f(contents, l1c, l2c, l3c)


def kernel(sdrs, contents, l1_sdr_bank, l1_content_bank,
           l2_sdr_bank, l2_content_bank, l3_sdr_bank, l3_content_bank):
    sdrs = jax.lax.stop_gradient(sdrs)
    contents = jax.lax.stop_gradient(contents)

    o1c, o2c, o3c = _sc_content_copy(
        contents, l1_content_bank, l2_content_bank, l3_content_bank)

    o1s = _tc_copy(sdrs)
    o2s = _tc_concat(l1_sdr_bank, l2_sdr_bank, rows=L2_CAP, a_rows=N, b_row0=N)
    o3s = _tc_concat(l2_sdr_bank, l3_sdr_bank, rows=L3_CAP, a_rows=N, b_row0=N)
    return (o1s, o1c, o2s, o2c, o3s, o3c)
